# baseline (speedup 1.0000x reference)
import jax
import jax.numpy as jnp
from jax.experimental import pallas as pl
from jax.experimental.pallas import tpu as pltpu

_BI = 200


def _pass_a_kernel(x_ref, w1_ref, adj_ref, w2_ref, h2_ref, h0_scr):
    i = pl.program_id(0)

    @pl.when(i == 0)
    def _():
        h0 = jax.lax.dot_general(
            x_ref[...], w1_ref[...], (((1,), (1,)), ((), ())),
            preferred_element_type=jnp.float32,
            precision=jax.lax.Precision.HIGHEST,
        )
        h0_scr[...] = h0.astype(jnp.bfloat16)

    adj_b = adj_ref[...].astype(jnp.bfloat16)
    h1 = jnp.dot(adj_b, h0_scr[...], preferred_element_type=jnp.float32)
    h2 = jax.lax.dot_general(
        jnp.maximum(h1, 0.0), w2_ref[...], (((1,), (1,)), ((), ())),
        preferred_element_type=jnp.float32,
        precision=jax.lax.Precision.HIGHEST,
    )
    h2_ref[...] = h2.astype(jnp.bfloat16)


def _pass_b_kernel(h2_ref, adj_ref, out_ref):
    adj_b = adj_ref[...].astype(jnp.bfloat16)
    o = jnp.dot(adj_b, h2_ref[...], preferred_element_type=jnp.float32)
    m = jnp.max(o, axis=1, keepdims=True)
    lse = jnp.log(jnp.sum(jnp.exp(o - m), axis=1, keepdims=True))
    out_ref[...] = o - m - lse


def kernel(x, adj, W1, W2):
    n, in_dim = x.shape
    hid_dim = W1.shape[0]
    out_dim = W2.shape[0]
    ni = n // _BI

    h2 = pl.pallas_call(
        _pass_a_kernel,
        grid=(ni,),
        in_specs=[
            pl.BlockSpec((n, in_dim), lambda i: (0, 0)),
            pl.BlockSpec((hid_dim, in_dim), lambda i: (0, 0)),
            pl.BlockSpec((_BI, n), lambda i: (i, 0)),
            pl.BlockSpec((out_dim, hid_dim), lambda i: (0, 0)),
        ],
        out_specs=pl.BlockSpec((_BI, out_dim), lambda i: (i, 0)),
        out_shape=jax.ShapeDtypeStruct((n, out_dim), jnp.bfloat16),
        scratch_shapes=[pltpu.VMEM((n, hid_dim), jnp.bfloat16)],
    )(x, W1, adj, W2)

    out = pl.pallas_call(
        _pass_b_kernel,
        grid=(ni,),
        in_specs=[
            pl.BlockSpec((n, out_dim), lambda i: (0, 0)),
            pl.BlockSpec((_BI, n), lambda i: (i, 0)),
        ],
        out_specs=pl.BlockSpec((_BI, out_dim), lambda i: (i, 0)),
        out_shape=jax.ShapeDtypeStruct((n, out_dim), jnp.float32),
    )(h2, adj)
    return out


# speedup vs baseline: 1.0833x; 1.0833x over previous
import jax
import jax.numpy as jnp
from jax.experimental import pallas as pl
from jax.experimental.pallas import tpu as pltpu

_BI = 400


def _fused_kernel(x_ref, w1_ref, adj_ref, w2_ref, out_ref, h0_scr, h2_scr):
    p = pl.program_id(0)
    i = pl.program_id(1)

    @pl.when((p == 0) & (i == 0))
    def _():
        h0 = jax.lax.dot_general(
            x_ref[...].astype(jnp.bfloat16),
            w1_ref[...].astype(jnp.bfloat16),
            (((1,), (1,)), ((), ())),
            preferred_element_type=jnp.float32,
        )
        h0_scr[...] = h0.astype(jnp.bfloat16)

    adj_b = adj_ref[...].astype(jnp.bfloat16)

    @pl.when(p == 0)
    def _():
        h1 = jnp.dot(adj_b, h0_scr[...], preferred_element_type=jnp.float32)
        h2 = jax.lax.dot_general(
            jnp.maximum(h1, 0.0),
            w2_ref[...].astype(jnp.bfloat16),
            (((1,), (1,)), ((), ())),
            preferred_element_type=jnp.float32,
        )
        h2_scr[pl.ds(i * _BI, _BI), :] = h2.astype(jnp.bfloat16)

    @pl.when(p == 1)
    def _():
        o = jnp.dot(adj_b, h2_scr[...], preferred_element_type=jnp.float32)
        m = jnp.max(o, axis=1, keepdims=True)
        lse = jnp.log(jnp.sum(jnp.exp(o - m), axis=1, keepdims=True))
        out_ref[...] = o - m - lse


def kernel(x, adj, W1, W2):
    n, in_dim = x.shape
    hid_dim = W1.shape[0]
    out_dim = W2.shape[0]
    ni = n // _BI

    return pl.pallas_call(
        _fused_kernel,
        grid=(2, ni),
        in_specs=[
            pl.BlockSpec((n, in_dim), lambda p, i: (0, 0)),
            pl.BlockSpec((hid_dim, in_dim), lambda p, i: (0, 0)),
            pl.BlockSpec((_BI, n), lambda p, i: (i, 0)),
            pl.BlockSpec((out_dim, hid_dim), lambda p, i: (0, 0)),
        ],
        out_specs=pl.BlockSpec((_BI, out_dim), lambda p, i: (i, 0)),
        out_shape=jax.ShapeDtypeStruct((n, out_dim), jnp.float32),
        scratch_shapes=[
            pltpu.VMEM((n, hid_dim), jnp.bfloat16),
            pltpu.VMEM((n, out_dim), jnp.bfloat16),
        ],
    )(x, W1, adj, W2)
